# baseline (speedup 1.0000x reference)
import functools

import jax
import jax.numpy as jnp
from jax import lax
from jax.experimental import pallas as pl
from jax.experimental.pallas import tpu as pltpu
from jax.experimental.pallas import tpu_sc as plsc

NUM_WORKERS = 32
BC = 128
LANES = 16


def _emb_lookup(table, idxT, H, NBT, D):
    mesh = plsc.VectorSubcoreMesh(core_axis_name="c", subcore_axis_name="s")
    n_tasks = H * NBT
    tpw = n_tasks // NUM_WORKERS
    DT = D // 8

    @functools.partial(
        pl.kernel,
        mesh=mesh,
        out_type=jax.ShapeDtypeStruct((H, DT, NBT, 8, BC), jnp.float32),
        scratch_types=[
            pltpu.VMEM((tpw, BC), jnp.int32),
            pltpu.VMEM((BC, D), jnp.float32),
            pltpu.VMEM((BC, D), jnp.float32),
            pltpu.VMEM((D, BC), jnp.float32),
            pltpu.VMEM((D, BC), jnp.float32),
            pltpu.SemaphoreType.DMA((2,)),
            pltpu.SemaphoreType.DMA((2,)),
        ],
        compiler_params=pltpu.CompilerParams(
            use_tc_tiling_on_sc=False, needs_layout_passes=False
        ),
    )
    def emb(table_hbm, idx_hbm, out_hbm, idx_v, rows0, rows1, tr0, tr1,
            sem_g, sem_s):
        wid = lax.axis_index("s") * 2 + lax.axis_index("c")
        t0 = wid * tpw
        pltpu.sync_copy(idx_hbm.at[pl.ds(t0, tpw)], idx_v)

        lane = lax.iota(jnp.int32, LANES)

        def start_gather(t, rows, b):
            pltpu.make_async_copy(
                table_hbm.at[idx_v.at[t]], rows, sem_g.at[b]
            ).start()

        def wait_gather(rows, b):
            pltpu.make_async_copy(
                table_hbm.at[idx_v.at[0]], rows, sem_g.at[b]
            ).wait()

        def transpose(rows, tr):
            @plsc.parallel_loop(0, D, unroll=16)
            def _(dd):
                cid = jnp.full((LANES,), 0, jnp.int32) + dd
                for bg in range(BC // LANES):
                    rid = lane + (bg * LANES)
                    tr[dd, pl.ds(bg * LANES, LANES)] = plsc.load_gather(
                        rows, [rid, cid]
                    )

        def start_scatter(t, tr, b):
            tg = t0 + t
            h = tg // NBT
            bt = tg % NBT
            for dt in range(DT):
                pltpu.make_async_copy(
                    tr.at[pl.ds(dt * 8, 8)], out_hbm.at[h, dt, bt],
                    sem_s.at[b],
                ).start()

        def wait_scatter(tr, b):
            for dt in range(DT):
                pltpu.make_async_copy(
                    tr.at[pl.ds(0, 8)], out_hbm.at[0, 0, 0], sem_s.at[b]
                ).wait()

        start_gather(0, rows0, 0)
        start_gather(1, rows1, 1)

        def group(g, carry):
            for b, rows, tr in ((0, rows0, tr0), (1, rows1, tr1)):
                t = g * 2 + b
                wait_gather(rows, b)

                @pl.when(g >= 1)
                def _():
                    wait_scatter(tr, b)

                transpose(rows, tr)
                start_scatter(t, tr, b)
                start_gather(jnp.minimum(t + 2, tpw - 1), rows, b)
            return carry

        lax.fori_loop(0, tpw // 2, group, 0)
        wait_gather(rows0, 0)
        wait_gather(rows1, 1)
        wait_scatter(tr0, 0)
        wait_scatter(tr1, 1)

    return emb(table, idxT)


def kernel(x, item_emb_weight):
    B, H = x.shape
    V, D = item_emb_weight.shape
    NBT = B // BC
    idxT = jnp.transpose(x).reshape(H * NBT, BC).astype(jnp.int32)
    Y = _emb_lookup(item_emb_weight, idxT, H, NBT, D)
    return Y.transpose(2, 4, 0, 1, 3).reshape(B, H, D)


# speedup vs baseline: 1.6292x; 1.6292x over previous
import functools

import jax
import jax.numpy as jnp
from jax import lax
from jax.experimental import pallas as pl
from jax.experimental.pallas import tpu as pltpu
from jax.experimental.pallas import tpu_sc as plsc

NUM_WORKERS = 32
BC = 128
LANES = 16


def _emb_lookup(table, idxT, H, NBT, D):
    mesh = plsc.VectorSubcoreMesh(core_axis_name="c", subcore_axis_name="s")
    n_tasks = H * NBT
    tpw = n_tasks // NUM_WORKERS
    DT = D // 8

    @functools.partial(
        pl.kernel,
        mesh=mesh,
        out_type=jax.ShapeDtypeStruct((H, DT, NBT, 8, BC), jnp.float32),
        scratch_types=[
            pltpu.VMEM((tpw, BC), jnp.int32),
            pltpu.VMEM((BC, D), jnp.float32),
            pltpu.VMEM((BC, D), jnp.float32),
            pltpu.VMEM((D, BC), jnp.float32),
            pltpu.VMEM((D, BC), jnp.float32),
            pltpu.SemaphoreType.DMA((2,)),
            pltpu.SemaphoreType.DMA((2,)),
        ],
        compiler_params=pltpu.CompilerParams(
            use_tc_tiling_on_sc=False, needs_layout_passes=False
        ),
    )
    def emb(table_hbm, idx_hbm, out_hbm, idx_v, rows0, rows1, tr0, tr1,
            sem_g, sem_s):
        wid = lax.axis_index("s") * 2 + lax.axis_index("c")
        t0 = wid * tpw
        pltpu.sync_copy(idx_hbm.at[pl.ds(t0, tpw)], idx_v)

        lane = lax.iota(jnp.int32, LANES)

        def start_gather(t, rows, b):
            pltpu.make_async_copy(
                table_hbm.at[idx_v.at[t]], rows, sem_g.at[b]
            ).start()

        def wait_gather(rows, b):
            pltpu.make_async_copy(
                table_hbm.at[idx_v.at[0]], rows, sem_g.at[b]
            ).wait()

        def transpose(rows, tr):
            @plsc.parallel_loop(0, D, unroll=8)
            def _(dd):
                cdiag = jnp.bitwise_and(dd + lane, D - 1)
                for bg in range(BC // LANES):
                    rid = lane + (bg * LANES)
                    v = plsc.load_gather(rows, [rid, cdiag])
                    plsc.store_scatter(tr, [cdiag, rid], v)

        def start_scatter(t, tr, b):
            tg = t0 + t
            h = tg // NBT
            bt = tg % NBT
            for dt in range(DT):
                pltpu.make_async_copy(
                    tr.at[pl.ds(dt * 8, 8)], out_hbm.at[h, dt, bt],
                    sem_s.at[b],
                ).start()

        def wait_scatter(tr, b):
            for dt in range(DT):
                pltpu.make_async_copy(
                    tr.at[pl.ds(0, 8)], out_hbm.at[0, 0, 0], sem_s.at[b]
                ).wait()

        start_gather(0, rows0, 0)
        start_gather(1, rows1, 1)

        def group(g, carry):
            for b, rows, tr in ((0, rows0, tr0), (1, rows1, tr1)):
                t = g * 2 + b
                wait_gather(rows, b)

                @pl.when(g >= 1)
                def _():
                    wait_scatter(tr, b)

                transpose(rows, tr)
                start_scatter(t, tr, b)
                start_gather(jnp.minimum(t + 2, tpw - 1), rows, b)
            return carry

        lax.fori_loop(0, tpw // 2, group, 0)
        wait_gather(rows0, 0)
        wait_gather(rows1, 1)
        wait_scatter(tr0, 0)
        wait_scatter(tr1, 1)

    return emb(table, idxT)


def kernel(x, item_emb_weight):
    B, H = x.shape
    V, D = item_emb_weight.shape
    NBT = B // BC
    idxT = jnp.transpose(x).reshape(H * NBT, BC).astype(jnp.int32)
    Y = _emb_lookup(item_emb_weight, idxT, H, NBT, D)
    return Y.transpose(2, 4, 0, 1, 3).reshape(B, H, D)


# speedup vs baseline: 1.6378x; 1.0053x over previous
import functools

import jax
import jax.numpy as jnp
from jax import lax
from jax.experimental import pallas as pl
from jax.experimental.pallas import tpu as pltpu
from jax.experimental.pallas import tpu_sc as plsc

NUM_WORKERS = 32
BC = 128
LANES = 16


def _emb_lookup(table, idxT, H, NBT, D):
    mesh = plsc.VectorSubcoreMesh(core_axis_name="c", subcore_axis_name="s")
    n_tasks = H * NBT
    tpw = n_tasks // NUM_WORKERS
    DT = D // 8

    @functools.partial(
        pl.kernel,
        mesh=mesh,
        out_type=jax.ShapeDtypeStruct((H, DT, NBT, 8, BC), jnp.float32),
        scratch_types=[
            pltpu.VMEM((tpw, BC), jnp.int32),
            pltpu.VMEM((BC, 2 * D), jnp.float32),
            pltpu.VMEM((BC, 2 * D), jnp.float32),
            pltpu.VMEM((D, BC), jnp.float32),
            pltpu.VMEM((D, BC), jnp.float32),
            pltpu.SemaphoreType.DMA((2,)),
            pltpu.SemaphoreType.DMA((2,)),
        ],
        compiler_params=pltpu.CompilerParams(
            use_tc_tiling_on_sc=False, needs_layout_passes=False
        ),
    )
    def emb(table_hbm, idx_hbm, out_hbm, idx_v, rows0, rows1, tr0, tr1,
            sem_g, sem_s):
        wid = lax.axis_index("s") * 2 + lax.axis_index("c")
        t0 = wid * tpw
        pltpu.sync_copy(idx_hbm.at[pl.ds(t0, tpw)], idx_v)

        lane = lax.iota(jnp.int32, LANES)

        def start_gather(t, rows, b):
            pltpu.make_async_copy(
                table_hbm.at[idx_v.at[t]], rows, sem_g.at[b]
            ).start()

        def wait_gather(rows, b):
            pltpu.make_async_copy(
                table_hbm.at[idx_v.at[0]], rows, sem_g.at[b]
            ).wait()

        def transpose(rows, tr):
            @plsc.parallel_loop(0, D, unroll=8)
            def _(dd):
                cdiag = jnp.bitwise_and(dd + lane, D - 1)
                for bg in range(BC // LANES):
                    rid = lane + (bg * LANES)
                    v = plsc.load_gather(rows, [rid, cdiag])
                    plsc.store_scatter(tr, [cdiag, rid], v)

        def start_scatter(t, tr, b):
            tg = t0 + t
            h = tg // NBT
            bt = tg % NBT
            for dt in range(DT):
                pltpu.make_async_copy(
                    tr.at[pl.ds(dt * 8, 8)], out_hbm.at[h, dt, bt],
                    sem_s.at[b],
                ).start()

        def wait_scatter(tr, b):
            for dt in range(DT):
                pltpu.make_async_copy(
                    tr.at[pl.ds(0, 8)], out_hbm.at[0, 0, 0], sem_s.at[b]
                ).wait()

        start_gather(0, rows0, 0)
        start_gather(1, rows1, 1)

        def group(g, carry):
            for b, rows, tr in ((0, rows0, tr0), (1, rows1, tr1)):
                t = g * 2 + b
                wait_gather(rows, b)

                @pl.when(g >= 1)
                def _():
                    wait_scatter(tr, b)

                transpose(rows, tr)
                start_scatter(t, tr, b)
                start_gather(jnp.minimum(t + 2, tpw - 1), rows, b)
            return carry

        lax.fori_loop(0, tpw // 2, group, 0)
        wait_gather(rows0, 0)
        wait_gather(rows1, 1)
        wait_scatter(tr0, 0)
        wait_scatter(tr1, 1)

    return emb(table, idxT)


def kernel(x, item_emb_weight):
    B, H = x.shape
    V, D = item_emb_weight.shape
    NBT = B // BC
    idxT = jnp.transpose(x).reshape(H * NBT, BC).astype(jnp.int32)
    tableP = jnp.pad(item_emb_weight, ((0, 0), (0, D)))
    Y = _emb_lookup(tableP, idxT, H, NBT, D)
    return Y.transpose(2, 4, 0, 1, 3).reshape(B, H, D)


# speedup vs baseline: 2.8725x; 1.7539x over previous
import functools

import jax
import jax.numpy as jnp
from jax import lax
from jax.experimental import pallas as pl
from jax.experimental.pallas import tpu as pltpu
from jax.experimental.pallas import tpu_sc as plsc

NUM_WORKERS = 32
BC = 128
LANES = 16


def _repack_table(tableT, V, D):
    mesh = plsc.VectorSubcoreMesh(core_axis_name="c", subcore_axis_name="s")
    n_blocks = V // BC
    tail = V - n_blocks * BC
    bpw = -(-n_blocks // NUM_WORKERS)
    bpw += bpw % 2

    @functools.partial(
        pl.kernel,
        mesh=mesh,
        out_type=jax.ShapeDtypeStruct((V // 2, 2 * D), jnp.float32),
        scratch_types=[
            pltpu.VMEM((D // 8, 8, BC), jnp.float32),
            pltpu.VMEM((D // 8, 8, BC), jnp.float32),
            pltpu.VMEM((BC // 16, 8, 2 * D), jnp.float32),
            pltpu.VMEM((BC // 16, 8, 2 * D), jnp.float32),
            pltpu.SemaphoreType.DMA((2,)),
            pltpu.SemaphoreType.DMA((2,)),
        ],
        compiler_params=pltpu.CompilerParams(needs_layout_passes=False),
    )
    def repack(tin, tout, buf0, buf1, trb0, trb1, sem_g, sem_s):
        wid = lax.axis_index("s") * 2 + lax.axis_index("c")
        lane = lax.iota(jnp.int32, LANES)

        def blk(j):
            return jnp.minimum(wid + NUM_WORKERS * j, n_blocks - 1)

        def start_read(c, buf, b):
            for dt in range(D // 8):
                pltpu.make_async_copy(
                    tin.at[pl.ds(dt * 8, 8), pl.ds(c * BC, BC)],
                    buf.at[dt], sem_g.at[b],
                ).start()

        def wait_read(buf, b):
            for dt in range(D // 8):
                pltpu.make_async_copy(
                    tin.at[pl.ds(0, 8), pl.ds(0, BC)],
                    buf.at[0], sem_g.at[b],
                ).wait()

        def transpose_blk(buf, trb, ngroups):
            @plsc.parallel_loop(0, D, unroll=8)
            def _(dd):
                ddiag = jnp.bitwise_and(dd + lane, D - 1)
                for bg in range(ngroups):
                    bcv = lane + (bg * LANES)
                    v = plsc.load_gather(
                        buf, [ddiag >> 3, ddiag & 7, bcv]
                    )
                    p = bcv >> 1
                    plsc.store_scatter(
                        trb,
                        [p >> 3, p & 7, (bcv & 1) * D + ddiag],
                        v,
                    )

        def start_write(c, trb, b):
            for pt in range(BC // 16):
                pltpu.make_async_copy(
                    trb.at[pt],
                    tout.at[pl.ds(c * (BC // 2) + pt * 8, 8)],
                    sem_s.at[b],
                ).start()

        def wait_write(trb, b):
            for pt in range(BC // 16):
                pltpu.make_async_copy(
                    trb.at[0], tout.at[pl.ds(0, 8)], sem_s.at[b]
                ).wait()

        start_read(blk(0), buf0, 0)
        start_read(blk(1), buf1, 1)

        def group(g, carry):
            for b, buf, trb in ((0, buf0, trb0), (1, buf1, trb1)):
                j = g * 2 + b
                wait_read(buf, b)

                @pl.when(g >= 1)
                def _():
                    wait_write(trb, b)

                transpose_blk(buf, trb, BC // LANES)
                start_write(blk(j), trb, b)
                start_read(blk(jnp.minimum(j + 2, bpw - 1)), buf, b)
            return carry

        lax.fori_loop(0, bpw // 2, group, 0)
        wait_read(buf0, 0)
        wait_read(buf1, 1)
        wait_write(trb0, 0)
        wait_write(trb1, 1)

    del tail
    return repack(tableT)


def _emb_lookup(table, idxT, H, NBT, D):
    mesh = plsc.VectorSubcoreMesh(core_axis_name="c", subcore_axis_name="s")
    n_tasks = H * NBT
    tpw = n_tasks // NUM_WORKERS
    DT = D // 8

    @functools.partial(
        pl.kernel,
        mesh=mesh,
        out_type=jax.ShapeDtypeStruct((H, DT, NBT, 8, BC), jnp.float32),
        scratch_types=[
            pltpu.VMEM((tpw, BC), jnp.int32),
            pltpu.VMEM((BC, D), jnp.float32),
            pltpu.VMEM((BC, D), jnp.float32),
            pltpu.VMEM((D, BC), jnp.float32),
            pltpu.VMEM((D, BC), jnp.float32),
            pltpu.SemaphoreType.DMA((2,)),
            pltpu.SemaphoreType.DMA((2,)),
        ],
        compiler_params=pltpu.CompilerParams(
            use_tc_tiling_on_sc=False, needs_layout_passes=False
        ),
    )
    def emb(table_hbm, idx_hbm, out_hbm, idx_v, rows0, rows1, tr0, tr1,
            sem_g, sem_s):
        wid = lax.axis_index("s") * 2 + lax.axis_index("c")
        t0 = wid * tpw
        pltpu.sync_copy(idx_hbm.at[pl.ds(t0, tpw)], idx_v)

        lane = lax.iota(jnp.int32, LANES)

        def start_gather(t, rows, b):
            pltpu.make_async_copy(
                table_hbm.at[idx_v.at[t]], rows, sem_g.at[b]
            ).start()

        def wait_gather(rows, b):
            pltpu.make_async_copy(
                table_hbm.at[idx_v.at[0]], rows, sem_g.at[b]
            ).wait()

        def transpose(rows, tr):
            @plsc.parallel_loop(0, D, unroll=8)
            def _(dd):
                cdiag = jnp.bitwise_and(dd + lane, D - 1)
                for bg in range(BC // LANES):
                    rid = lane + (bg * LANES)
                    v = plsc.load_gather(rows, [rid, cdiag])
                    plsc.store_scatter(tr, [cdiag, rid], v)

        def start_scatter(t, tr, b):
            tg = t0 + t
            h = tg // NBT
            bt = tg % NBT
            for dt in range(DT):
                pltpu.make_async_copy(
                    tr.at[pl.ds(dt * 8, 8)], out_hbm.at[h, dt, bt],
                    sem_s.at[b],
                ).start()

        def wait_scatter(tr, b):
            for dt in range(DT):
                pltpu.make_async_copy(
                    tr.at[pl.ds(0, 8)], out_hbm.at[0, 0, 0], sem_s.at[b]
                ).wait()

        start_gather(0, rows0, 0)
        start_gather(1, rows1, 1)

        def group(g, carry):
            for b, rows, tr in ((0, rows0, tr0), (1, rows1, tr1)):
                t = g * 2 + b
                wait_gather(rows, b)

                @pl.when(g >= 1)
                def _():
                    wait_scatter(tr, b)

                transpose(rows, tr)
                start_scatter(t, tr, b)
                start_gather(jnp.minimum(t + 2, tpw - 1), rows, b)
            return carry

        lax.fori_loop(0, tpw // 2, group, 0)
        wait_gather(rows0, 0)
        wait_gather(rows1, 1)
        wait_scatter(tr0, 0)
        wait_scatter(tr1, 1)

    return emb(table, idxT)


def kernel(x, item_emb_weight):
    B, H = x.shape
    V, D = item_emb_weight.shape
    NBT = B // BC
    idxT = jnp.transpose(x).reshape(H * NBT, BC).astype(jnp.int32)
    tableP = _repack_table(jnp.transpose(item_emb_weight), V, D)
    tail = V - (V // BC) * BC
    tail_rows = item_emb_weight[V - tail:, :].reshape(tail // 2, 2 * D)
    tableP = lax.dynamic_update_slice(
        tableP, tail_rows, ((V - tail) // 2, 0)
    )
    Y = _emb_lookup(tableP.reshape(V, D), idxT, H, NBT, D)
    return Y.transpose(2, 4, 0, 1, 3).reshape(B, H, D)


# speedup vs baseline: 3.0378x; 1.0575x over previous
import functools

import jax
import jax.numpy as jnp
from jax import lax
from jax.experimental import pallas as pl
from jax.experimental.pallas import tpu as pltpu
from jax.experimental.pallas import tpu_sc as plsc

NUM_WORKERS = 32
BC = 128
LANES = 16


def _repack_table(tableT, V, D):
    mesh = plsc.VectorSubcoreMesh(core_axis_name="c", subcore_axis_name="s")
    n_blocks = V // BC
    tail = V - n_blocks * BC
    bpw = -(-n_blocks // NUM_WORKERS)
    bpw += bpw % 2

    @functools.partial(
        pl.kernel,
        mesh=mesh,
        out_type=jax.ShapeDtypeStruct((V // 2, 2 * D), jnp.float32),
        scratch_types=[
            pltpu.VMEM((D // 8, 8, BC), jnp.float32),
            pltpu.VMEM((D // 8, 8, BC), jnp.float32),
            pltpu.VMEM((BC // 16, 8, 2 * D), jnp.float32),
            pltpu.VMEM((BC // 16, 8, 2 * D), jnp.float32),
            pltpu.SemaphoreType.DMA((2,)),
            pltpu.SemaphoreType.DMA((2,)),
        ],
        compiler_params=pltpu.CompilerParams(needs_layout_passes=False),
    )
    def repack(tin, tout, buf0, buf1, trb0, trb1, sem_g, sem_s):
        wid = lax.axis_index("s") * 2 + lax.axis_index("c")
        lane = lax.iota(jnp.int32, LANES)

        def blk(j):
            return jnp.minimum(wid + NUM_WORKERS * j, n_blocks - 1)

        def start_read(c, buf, b):
            for dt in range(D // 8):
                pltpu.make_async_copy(
                    tin.at[pl.ds(dt * 8, 8), pl.ds(c * BC, BC)],
                    buf.at[dt], sem_g.at[b],
                ).start()

        def wait_read(buf, b):
            for dt in range(D // 8):
                pltpu.make_async_copy(
                    tin.at[pl.ds(0, 8), pl.ds(0, BC)],
                    buf.at[0], sem_g.at[b],
                ).wait()

        def transpose_blk(buf, trb, ngroups):
            @plsc.parallel_loop(0, D, unroll=4)
            def _(dd):
                ddiag = jnp.bitwise_and(dd + lane, D - 1)
                for bg in range(ngroups):
                    bcv = lane + (bg * LANES)
                    v = plsc.load_gather(
                        buf, [ddiag >> 3, ddiag & 7, bcv]
                    )
                    p = bcv >> 1
                    plsc.store_scatter(
                        trb,
                        [p >> 3, p & 7, (bcv & 1) * D + ddiag],
                        v,
                    )

        def start_write(c, trb, b):
            for pt in range(BC // 16):
                pltpu.make_async_copy(
                    trb.at[pt],
                    tout.at[pl.ds(c * (BC // 2) + pt * 8, 8)],
                    sem_s.at[b],
                ).start()

        def wait_write(trb, b):
            for pt in range(BC // 16):
                pltpu.make_async_copy(
                    trb.at[0], tout.at[pl.ds(0, 8)], sem_s.at[b]
                ).wait()

        start_read(blk(0), buf0, 0)
        start_read(blk(1), buf1, 1)

        def group(g, carry):
            for b, buf, trb in ((0, buf0, trb0), (1, buf1, trb1)):
                j = g * 2 + b
                wait_read(buf, b)

                @pl.when(g >= 1)
                def _():
                    wait_write(trb, b)

                transpose_blk(buf, trb, BC // LANES)
                start_write(blk(j), trb, b)
                start_read(blk(jnp.minimum(j + 2, bpw - 1)), buf, b)
            return carry

        lax.fori_loop(0, bpw // 2, group, 0)
        wait_read(buf0, 0)
        wait_read(buf1, 1)
        wait_write(trb0, 0)
        wait_write(trb1, 1)

    del tail
    return repack(tableT)


def _emb_lookup(table, idxT, H, NBT, D):
    mesh = plsc.VectorSubcoreMesh(core_axis_name="c", subcore_axis_name="s")
    n_tasks = H * NBT
    tpw = n_tasks // NUM_WORKERS
    DT = D // 8

    @functools.partial(
        pl.kernel,
        mesh=mesh,
        out_type=jax.ShapeDtypeStruct((H, DT, NBT, 8, BC), jnp.float32),
        scratch_types=[
            pltpu.VMEM((tpw, BC), jnp.int32),
            pltpu.VMEM((BC, D), jnp.float32),
            pltpu.VMEM((BC, D), jnp.float32),
            pltpu.VMEM((D, BC), jnp.float32),
            pltpu.VMEM((D, BC), jnp.float32),
            pltpu.SemaphoreType.DMA((2,)),
            pltpu.SemaphoreType.DMA((2,)),
        ],
        compiler_params=pltpu.CompilerParams(
            use_tc_tiling_on_sc=False, needs_layout_passes=False
        ),
    )
    def emb(table_hbm, idx_hbm, out_hbm, idx_v, rows0, rows1, tr0, tr1,
            sem_g, sem_s):
        wid = lax.axis_index("s") * 2 + lax.axis_index("c")
        t0 = wid * tpw
        pltpu.sync_copy(idx_hbm.at[pl.ds(t0, tpw)], idx_v)

        lane = lax.iota(jnp.int32, LANES)

        def start_gather(t, rows, b):
            pltpu.make_async_copy(
                table_hbm.at[idx_v.at[t]], rows, sem_g.at[b]
            ).start()

        def wait_gather(rows, b):
            pltpu.make_async_copy(
                table_hbm.at[idx_v.at[0]], rows, sem_g.at[b]
            ).wait()

        def transpose(rows, tr):
            @plsc.parallel_loop(0, D, unroll=4)
            def _(dd):
                cdiag = jnp.bitwise_and(dd + lane, D - 1)
                for bg in range(BC // LANES):
                    rid = lane + (bg * LANES)
                    v = plsc.load_gather(rows, [rid, cdiag])
                    plsc.store_scatter(tr, [cdiag, rid], v)

        def start_scatter(t, tr, b):
            tg = t0 + t
            h = tg // NBT
            bt = tg % NBT
            for dt in range(DT):
                pltpu.make_async_copy(
                    tr.at[pl.ds(dt * 8, 8)], out_hbm.at[h, dt, bt],
                    sem_s.at[b],
                ).start()

        def wait_scatter(tr, b):
            for dt in range(DT):
                pltpu.make_async_copy(
                    tr.at[pl.ds(0, 8)], out_hbm.at[0, 0, 0], sem_s.at[b]
                ).wait()

        start_gather(0, rows0, 0)
        start_gather(1, rows1, 1)

        def group(g, carry):
            for b, rows, tr in ((0, rows0, tr0), (1, rows1, tr1)):
                t = g * 2 + b
                wait_gather(rows, b)

                @pl.when(g >= 1)
                def _():
                    wait_scatter(tr, b)

                transpose(rows, tr)
                start_scatter(t, tr, b)
                start_gather(jnp.minimum(t + 2, tpw - 1), rows, b)
            return carry

        lax.fori_loop(0, tpw // 2, group, 0)
        wait_gather(rows0, 0)
        wait_gather(rows1, 1)
        wait_scatter(tr0, 0)
        wait_scatter(tr1, 1)

    return emb(table, idxT)


def kernel(x, item_emb_weight):
    B, H = x.shape
    V, D = item_emb_weight.shape
    NBT = B // BC
    idxT = jnp.transpose(x).reshape(H * NBT, BC).astype(jnp.int32)
    tableP = _repack_table(jnp.transpose(item_emb_weight), V, D)
    tail = V - (V // BC) * BC
    tail_rows = item_emb_weight[V - tail:, :].reshape(tail // 2, 2 * D)
    tableP = lax.dynamic_update_slice(
        tableP, tail_rows, ((V - tail) // 2, 0)
    )
    Y = _emb_lookup(tableP.reshape(V, D), idxT, H, NBT, D)
    return Y.transpose(2, 4, 0, 1, 3).reshape(B, H, D)
